# baseline (speedup 1.0000x reference)
import functools
import math

import jax
import jax.numpy as jnp
from jax import lax
from jax.experimental import pallas as pl
from jax.experimental.pallas import tpu as pltpu
from jax.experimental.pallas import tpu_sc as plsc

EMB = 64
SCALE = math.sqrt(EMB)
NC = 2
NS = 16
NW = NC * NS
L = 16


def _body(seq, tok_hbm, table_hbm, out_hbm,
          traw, idx2, gbuf, obuf, g_sems, s_sems):
    wid = lax.axis_index("s") * NC + lax.axis_index("c")
    b0 = wid * 128
    n_blocks = seq // 8

    def stage_block(blk, bb):
        pltpu.sync_copy(tok_hbm.at[pl.ds(blk * 8, 8), pl.ds(b0, 128)],
                        traw.at[bb])

        @plsc.parallel_loop(0, 8 * 128 // L, unroll=4)
        def _(i):
            r = i // (128 // L)
            sl = pl.ds((i % (128 // L)) * L, L)
            idx2[bb, r, sl] = lax.shift_right_logical(traw[bb, r, sl], 1)

    def start_gather(s, gb):
        bb = lax.rem(s // 8, 2)
        pltpu.async_copy(
            table_hbm.at[idx2.at[bb, lax.rem(s, 8)]],
            gbuf.at[gb],
            g_sems.at[gb],
        )

    def wait_gather(s, gb):
        bb = lax.rem(s // 8, 2)
        pltpu.make_async_copy(
            table_hbm.at[idx2.at[bb, lax.rem(s, 8)]],
            gbuf.at[gb],
            g_sems.at[gb],
        ).wait()

    def wait_store(s, gb):
        pltpu.make_async_copy(
            obuf.at[gb],
            out_hbm.at[s, :, pl.ds(b0, 128)],
            s_sems.at[gb],
        ).wait()

    stage_block(0, 0)
    start_gather(0, 0)

    def s_body(s, carry):
        gb = lax.rem(s, 2)
        nb = lax.rem(s + 1, 2)

        @pl.when(s + 1 < seq)
        def _():
            @pl.when(lax.rem(s + 1, 8) == 0)
            def _():
                stage_block((s + 1) // 8, lax.rem((s + 1) // 8, 2))

            start_gather(s + 1, nb)

        wait_gather(s, gb)

        bb = lax.rem(s // 8, 2)
        s8 = lax.rem(s, 8)
        iota = lax.iota(jnp.int32, L)
        rows = []
        cols = []
        for g in range(128 // L):
            t = traw[bb, s8, pl.ds(g * L, L)]
            rows.append(iota + (g * L))
            cols.append(lax.rem(t, 2) * EMB)

        @pl.when(s >= 2)
        def _():
            wait_store(s - 2, gb)

        @plsc.parallel_loop(0, EMB, unroll=2)
        def _(c):
            for g in range(128 // L):
                v = plsc.load_gather(gbuf.at[gb], [rows[g], cols[g] + c])
                obuf[gb, c, pl.ds(g * L, L)] = v * SCALE

        pltpu.async_copy(
            obuf.at[gb],
            out_hbm.at[s, :, pl.ds(b0, 128)],
            s_sems.at[gb],
        )
        return carry

    lax.fori_loop(0, seq, s_body, 0)
    wait_store(seq - 2, lax.rem(seq - 2, 2))
    wait_store(seq - 1, lax.rem(seq - 1, 2))


def kernel(tokens, embedding_weight):
    b, s = tokens.shape
    v, e = embedding_weight.shape
    assert b == NW * 128 and e == EMB and s % 8 == 0

    tokens_t = tokens.T.astype(jnp.int32)
    table2 = embedding_weight.reshape(v // 2, 2 * e)

    mesh = plsc.VectorSubcoreMesh(core_axis_name="c", subcore_axis_name="s")
    run = pl.kernel(
        functools.partial(_body, s),
        mesh=mesh,
        out_type=jax.ShapeDtypeStruct((s, EMB, b), jnp.float32),
        scratch_types=[
            pltpu.VMEM((2, 8, 128), jnp.int32),
            pltpu.VMEM((2, 8, 128), jnp.int32),
            pltpu.VMEM((2, 128, 2 * EMB), jnp.float32),
            pltpu.VMEM((2, EMB, 128), jnp.float32),
            pltpu.SemaphoreType.DMA((2,)),
            pltpu.SemaphoreType.DMA((2,)),
        ],
        compiler_params=pltpu.CompilerParams(
            use_tc_tiling_on_sc=True, needs_layout_passes=False),
    )
    out = run(tokens_t, table2)
    return out.transpose(2, 0, 1)


# speedup vs baseline: 1.0342x; 1.0342x over previous
import functools
import math

import jax
import jax.numpy as jnp
from jax import lax
from jax.experimental import pallas as pl
from jax.experimental.pallas import tpu as pltpu
from jax.experimental.pallas import tpu_sc as plsc

EMB = 64
SCALE = math.sqrt(EMB)

NC = 2
NS = 16
NW = NC * NS

RPC = 2
NBUF = 3


def _emb_body(rows_per_tile, seq, tokens_flat_hbm, table_hbm,
              out_hbm, idx_v, rows_v, g_sems, s_sems):
    wid = lax.axis_index("s") * NC + lax.axis_index("c")
    base = wid * rows_per_tile
    n_chunks = rows_per_tile // RPC
    cn = RPC * seq

    pltpu.sync_copy(tokens_flat_hbm.at[pl.ds(base * seq, rows_per_tile * seq)],
                    idx_v)

    def start_gathers(c, buf):
        pltpu.async_copy(
            table_hbm.at[idx_v.at[pl.ds(c * cn, cn)]],
            rows_v.at[buf],
            g_sems.at[buf],
        )

    def drain_gathers(c, buf):
        pltpu.make_async_copy(
            table_hbm.at[idx_v.at[pl.ds(c * cn, cn)]],
            rows_v.at[buf],
            g_sems.at[buf],
        ).wait()

    def start_store(c, buf):
        for r in range(RPC):
            pltpu.async_copy(
                rows_v.at[buf, pl.ds(r * seq, seq)],
                out_hbm.at[base + c * RPC + r],
                s_sems.at[buf],
            )

    def wait_store(c, buf):
        for r in range(RPC):
            pltpu.make_async_copy(
                rows_v.at[buf, pl.ds(r * seq, seq)],
                out_hbm.at[base + c * RPC + r],
                s_sems.at[buf],
            ).wait()

    start_gathers(0, 0)
    start_gathers(1, 1)

    def chunk_body(c, carry):
        buf = lax.rem(c, NBUF)
        drain_gathers(c, buf)

        @plsc.parallel_loop(0, cn, unroll=8)
        def scale_row(r):
            vals = [rows_v[buf, r, pl.ds(j * 16, 16)] for j in range(EMB // 16)]
            for j in range(EMB // 16):
                rows_v[buf, r, pl.ds(j * 16, 16)] = vals[j] * SCALE

        start_store(c, buf)

        @pl.when(c + 2 < n_chunks)
        def _():
            nb = lax.rem(c + 2, NBUF)

            @pl.when(c >= 1)
            def _():
                wait_store(c - 1, nb)

            start_gathers(c + 2, nb)

        return carry

    lax.fori_loop(0, n_chunks, chunk_body, 0)

    for t in (3, 2, 1):
        c = n_chunks - t
        wait_store(c, lax.rem(c, NBUF))


def kernel(tokens, embedding_weight):
    b, s = tokens.shape
    assert b % (NW * RPC) == 0
    rows_per_tile = b // NW

    tokens_flat = tokens.astype(jnp.int32).reshape(b * s)

    mesh = plsc.VectorSubcoreMesh(core_axis_name="c", subcore_axis_name="s")
    run = pl.kernel(
        functools.partial(_emb_body, rows_per_tile, s),
        mesh=mesh,
        out_type=jax.ShapeDtypeStruct((b, s, EMB), jnp.float32),
        scratch_types=[
            pltpu.VMEM((rows_per_tile * s,), jnp.int32),
            pltpu.VMEM((NBUF, RPC * s, EMB), jnp.float32),
            pltpu.SemaphoreType.DMA((NBUF,)),
            pltpu.SemaphoreType.DMA((NBUF,)),
        ],
        compiler_params=pltpu.CompilerParams(use_tc_tiling_on_sc=False),
    )
    return run(tokens_flat, embedding_weight)
